# baseline (speedup 1.0000x reference)
import functools

import jax
import jax.numpy as jnp
from jax.experimental import pallas as pl
from jax.experimental.pallas import tpu as pltpu

_LOW_BOUND = 1
_LOG2E = 1.4426950408889634

_TR = 2048
_SUB = 512


def _fused_body(n, x_ref, w1_ref, b1_ref, w2_ref, b2_ref, g_ref,
                dec_ref, keep_ref):
    lane = jax.lax.broadcasted_iota(jnp.int32, (1, _SUB), 1)
    col0 = (lane % n) == 0
    for k in range(_TR // _SUB):
        xs = x_ref[k * _SUB:(k + 1) * _SUB, :]
        h = jnp.dot(xs, w1_ref[...], preferred_element_type=jnp.float32)
        h = jnp.maximum(h + b1_ref[...], 0.0)
        lt = jax.lax.dot_general(
            w2_ref[...], h, (((0,), (1,)), ((), ())),
            preferred_element_type=jnp.float32,
        )
        lt = lt + b2_ref[...]
        d = lt[1:2, :] - lt[0:1, :]
        dec = (d > 0.0).astype(jnp.float32)
        grp = jnp.dot(dec, g_ref[...], preferred_element_type=jnp.float32)
        dec = jnp.where((grp == 0.0) & col0, 1.0, dec)
        dec_ref[0, k:k + 1, :] = dec
        keep_ref[0, k:k + 1, :] = 1.0 / (1.0 + jnp.exp2(d * -_LOG2E))


@jax.jit
def kernel(slots, W1, b1, W2, b2, global_step):
    B, N, DIM = slots.shape
    F = W1.shape[1]
    x = slots.reshape(B * N, DIM)
    b1r = b1.reshape(1, F)
    b2r = b2.reshape(2, 1)
    lane = jnp.arange(_SUB, dtype=jnp.int32)
    gmat = (lane[:, None] // N == lane[None, :] // N).astype(jnp.float32)

    chunks = B * N // _SUB
    grid = (B * N // _TR,)
    out = pl.pallas_call(
        functools.partial(_fused_body, N),
        grid=grid,
        in_specs=[
            pl.BlockSpec((_TR, DIM), lambda i: (i, 0)),
            pl.BlockSpec((DIM, F), lambda i: (0, 0)),
            pl.BlockSpec((1, F), lambda i: (0, 0)),
            pl.BlockSpec((F, 2), lambda i: (0, 0)),
            pl.BlockSpec((2, 1), lambda i: (0, 0)),
            pl.BlockSpec((_SUB, _SUB), lambda i: (0, 0)),
        ],
        out_specs=[
            pl.BlockSpec((1, _TR // _SUB, _SUB), lambda i: (i, 0, 0)),
            pl.BlockSpec((1, _TR // _SUB, _SUB), lambda i: (i, 0, 0)),
        ],
        out_shape=[
            jax.ShapeDtypeStruct((chunks // (_TR // _SUB), _TR // _SUB, _SUB), jnp.float32),
            jax.ShapeDtypeStruct((chunks // (_TR // _SUB), _TR // _SUB, _SUB), jnp.float32),
        ],
        compiler_params=pltpu.CompilerParams(
            dimension_semantics=("arbitrary",),
        ),
    )(x, W1, b1r, W2, b2r, gmat)
    return (out[0].reshape(B, N), out[1].reshape(B, N))


# speedup vs baseline: 1.0619x; 1.0619x over previous
import functools

import jax
import jax.numpy as jnp
from jax.experimental import pallas as pl
from jax.experimental.pallas import tpu as pltpu

_LOW_BOUND = 1
_LOG2E = 1.4426950408889634

_TR = 2048
_SUB = 512


def _fused_body(n, x_ref, w1_ref, b1_ref, w2_ref, b2_ref,
                dec_ref, keep_ref):
    lane = jax.lax.broadcasted_iota(jnp.int32, (1, _SUB), 1)
    col0 = (lane % n) == 0
    for k in range(_TR // _SUB):
        xs = x_ref[k * _SUB:(k + 1) * _SUB, :]
        h = jnp.dot(xs, w1_ref[...], preferred_element_type=jnp.float32)
        h = jnp.maximum(h + b1_ref[...], 0.0)
        lt = jax.lax.dot_general(
            w2_ref[...], h, (((0,), (1,)), ((), ())),
            preferred_element_type=jnp.float32,
        )
        lt = lt + b2_ref[...]
        d = lt[1:2, :] - lt[0:1, :]
        dec = (d > 0.0).astype(jnp.float32)
        m = dec
        for s in (1, 2, 4, 8, 16, 32):
            m = jnp.maximum(m, jnp.roll(m, -s, axis=1))
        dec = jnp.where((m == 0.0) & col0, 1.0, dec)
        dec_ref[0, k:k + 1, :] = dec
        keep_ref[0, k:k + 1, :] = 1.0 / (1.0 + jnp.exp2(d * -_LOG2E))


@jax.jit
def kernel(slots, W1, b1, W2, b2, global_step):
    B, N, DIM = slots.shape
    F = W1.shape[1]
    x = slots.reshape(B * N, DIM)
    b1r = b1.reshape(1, F)
    b2r = b2.reshape(2, 1)

    chunks = B * N // _SUB
    grid = (B * N // _TR,)
    out = pl.pallas_call(
        functools.partial(_fused_body, N),
        grid=grid,
        in_specs=[
            pl.BlockSpec((_TR, DIM), lambda i: (i, 0)),
            pl.BlockSpec((DIM, F), lambda i: (0, 0)),
            pl.BlockSpec((1, F), lambda i: (0, 0)),
            pl.BlockSpec((F, 2), lambda i: (0, 0)),
            pl.BlockSpec((2, 1), lambda i: (0, 0)),
        ],
        out_specs=[
            pl.BlockSpec((1, _TR // _SUB, _SUB), lambda i: (i, 0, 0)),
            pl.BlockSpec((1, _TR // _SUB, _SUB), lambda i: (i, 0, 0)),
        ],
        out_shape=[
            jax.ShapeDtypeStruct((chunks // (_TR // _SUB), _TR // _SUB, _SUB), jnp.float32),
            jax.ShapeDtypeStruct((chunks // (_TR // _SUB), _TR // _SUB, _SUB), jnp.float32),
        ],
        compiler_params=pltpu.CompilerParams(
            dimension_semantics=("arbitrary",),
        ),
    )(x, W1, b1r, W2, b2r)
    return (out[0].reshape(B, N), out[1].reshape(B, N))


# speedup vs baseline: 1.2226x; 1.1514x over previous
import functools

import jax
import jax.numpy as jnp
from jax.experimental import pallas as pl
from jax.experimental.pallas import tpu as pltpu

_LOW_BOUND = 1
_LOG2E = 1.4426950408889634


def _fused_body(n, sub, x_ref, w1_ref, b1_ref, w2_ref, b2_ref, dec_ref, keep_ref):
    rows_total = x_ref.shape[0]
    for k in range(rows_total // sub):
        xs = x_ref[k * sub:(k + 1) * sub, :]
        h = jnp.dot(xs, w1_ref[...], preferred_element_type=jnp.float32)
        h = jnp.maximum(h + b1_ref[...], 0.0)
        logits = jnp.dot(h, w2_ref[...], preferred_element_type=jnp.float32)
        logits = logits + b2_ref[...]
        d = logits[:, 1:2] - logits[:, 0:1]
        rows = sub // n
        d = d.reshape(rows, n)
        dec = (d > 0.0).astype(jnp.float32)
        any_active = jnp.max(dec, axis=1, keepdims=True)
        col0 = jax.lax.broadcasted_iota(jnp.int32, dec.shape, 1) == 0
        dec = jnp.where((any_active == 0.0) & col0, 1.0, dec)
        dec_ref[k * rows:(k + 1) * rows, :] = dec
        e = jnp.exp2(d * -_LOG2E)
        keep_ref[k * rows:(k + 1) * rows, :] = 1.0 / (1.0 + e)


@jax.jit
def kernel(slots, W1, b1, W2, b2, global_step):
    B, N, DIM = slots.shape
    F = W1.shape[1]
    x = slots.reshape(B * N, DIM)
    b1r = b1.reshape(1, F)
    b2r = b2.reshape(1, 2)

    R = 2048
    SUB = 1024
    grid = (B * N // R,)
    out = pl.pallas_call(
        functools.partial(_fused_body, N, SUB),
        grid=grid,
        in_specs=[
            pl.BlockSpec((R, DIM), lambda i: (i, 0)),
            pl.BlockSpec((DIM, F), lambda i: (0, 0)),
            pl.BlockSpec((1, F), lambda i: (0, 0)),
            pl.BlockSpec((F, 2), lambda i: (0, 0)),
            pl.BlockSpec((1, 2), lambda i: (0, 0)),
        ],
        out_specs=[
            pl.BlockSpec((R // N, N), lambda i: (i, 0)),
            pl.BlockSpec((R // N, N), lambda i: (i, 0)),
        ],
        out_shape=[
            jax.ShapeDtypeStruct((B, N), jnp.float32),
            jax.ShapeDtypeStruct((B, N), jnp.float32),
        ],
        compiler_params=pltpu.CompilerParams(
            dimension_semantics=("arbitrary",),
        ),
    )(x, W1, b1r, W2, b2r)
    return (out[0], out[1])


# speedup vs baseline: 1.2316x; 1.0074x over previous
import functools

import jax
import jax.numpy as jnp
from jax.experimental import pallas as pl
from jax.experimental.pallas import tpu as pltpu

_LOW_BOUND = 1
_LOG2E = 1.4426950408889634


def _fused_body(n, sub, x_ref, w1_ref, b1_ref, w2_ref, b2_ref, dec_ref, keep_ref):
    rows_total = x_ref.shape[0]
    for k in range(rows_total // sub):
        xs = x_ref[k * sub:(k + 1) * sub, :]
        h = jnp.dot(xs, w1_ref[...], preferred_element_type=jnp.float32)
        h = jnp.maximum(h + b1_ref[...], 0.0)
        logits = jnp.dot(h, w2_ref[...], preferred_element_type=jnp.float32)
        logits = logits + b2_ref[...]
        d = logits[:, 1:2] - logits[:, 0:1]
        rows = sub // n
        d = d.reshape(rows, n)
        dec = (d > 0.0).astype(jnp.float32)
        any_active = jnp.max(dec, axis=1, keepdims=True)
        col0 = jax.lax.broadcasted_iota(jnp.int32, dec.shape, 1) == 0
        dec = jnp.where((any_active == 0.0) & col0, 1.0, dec)
        dec_ref[k * rows:(k + 1) * rows, :] = dec
        e = jnp.exp2(d * -_LOG2E)
        keep_ref[k * rows:(k + 1) * rows, :] = 1.0 / (1.0 + e)


@jax.jit
def kernel(slots, W1, b1, W2, b2, global_step):
    B, N, DIM = slots.shape
    F = W1.shape[1]
    x = slots.reshape(B * N, DIM)
    b1r = b1.reshape(1, F)
    b2r = b2.reshape(1, 2)

    R = 2048
    SUB = 512
    grid = (B * N // R,)
    out = pl.pallas_call(
        functools.partial(_fused_body, N, SUB),
        grid=grid,
        in_specs=[
            pl.BlockSpec((R, DIM), lambda i: (i, 0)),
            pl.BlockSpec((DIM, F), lambda i: (0, 0)),
            pl.BlockSpec((1, F), lambda i: (0, 0)),
            pl.BlockSpec((F, 2), lambda i: (0, 0)),
            pl.BlockSpec((1, 2), lambda i: (0, 0)),
        ],
        out_specs=[
            pl.BlockSpec((R // N, N), lambda i: (i, 0)),
            pl.BlockSpec((R // N, N), lambda i: (i, 0)),
        ],
        out_shape=[
            jax.ShapeDtypeStruct((B, N), jnp.float32),
            jax.ShapeDtypeStruct((B, N), jnp.float32),
        ],
        compiler_params=pltpu.CompilerParams(
            dimension_semantics=("arbitrary",),
        ),
    )(x, W1, b1r, W2, b2r)
    return (out[0], out[1])
